# baseline (speedup 1.0000x reference)
import functools

import jax
import jax.numpy as jnp
from jax import lax
from jax.experimental import pallas as pl
from jax.experimental.pallas import tpu as pltpu, tpu_sc as plsc

N = 10000
E = 320000
R = 5
D = 128
TS = 768

NRP = 51200
NP = NRP // R
QROWS = NRP // 4
QN = NP // 4
ACCROWS = 13312
CH = 80
CAPB = 10080
CAP = 10336
TRASH_OFF = 10320
TOT = 32 * CAP
BN = 400
GRID = N // BN
EPT = E // 32


def _leaky(x):
    return jnp.where(x >= 0, x, 0.01 * x)


def _vsum(v):
    return lax.reduce_sum_p.bind(v, axes=(0,))


def _isum(m):
    v = jnp.where(m, jnp.full((16,), 1, jnp.int32),
                  jnp.full((16,), 0, jnp.int32))
    return lax.reduce_sum_p.bind(v, axes=(0,))


def _sc_kernel(**kw):
    def deco(fn):
        @functools.wraps(fn)
        def call(*args):
            mesh = plsc.VectorSubcoreMesh(core_axis_name="c",
                                          subcore_axis_name="s")
            params = pltpu.CompilerParams(needs_layout_passes=False)
            return pl.kernel(fn, mesh=mesh, compiler_params=params,
                             **kw)(*args)
        return call
    return deco


@_sc_kernel(
    out_type=(
        jax.ShapeDtypeStruct((TOT,), jnp.int32),
        jax.ShapeDtypeStruct((TOT,), jnp.int32),
        jax.ShapeDtypeStruct((512,), jnp.int32),
    ),
    scratch_types=[
        pltpu.VMEM((EPT,), jnp.int32),
        pltpu.VMEM((EPT,), jnp.int32),
        pltpu.VMEM((EPT,), jnp.int32),
        pltpu.VMEM((CH,), jnp.int32),
        pltpu.VMEM((CH,), jnp.int32),
        pltpu.VMEM((CH,), jnp.int32),
        pltpu.VMEM((CH,), jnp.int32),
        pltpu.VMEM((CH,), jnp.int32),
        pltpu.VMEM((CH,), jnp.int32),
        pltpu.VMEM((16,), jnp.int32),
        pltpu.VMEM((16,), jnp.int32),
        pltpu.VMEM((16,), jnp.int32),
        pltpu.VMEM((16,), jnp.int32),
        pltpu.SemaphoreType.DMA,
        pltpu.SemaphoreType.DMA,
        pltpu.SemaphoreType.DMA,
        pltpu.SemaphoreType.DMA,
    ],
)
def _part(src_hbm, et_hbm, dst_hbm, gi_hbm, cbl_hbm, meta_hbm,
          sbufL, ebufL, dbufL, givb0, cblb0, posb0, givb1, cblb1, posb1,
          givp, cblp, posp, mbuf, semg0, semc0, semg1, semc1):
    c = lax.axis_index("c")
    s = lax.axis_index("s")
    w = s * 2 + c
    lanes = lax.iota(jnp.int32, 16)
    one = jnp.full((16,), 1, jnp.int32)
    zero = jnp.full((16,), 0, jnp.int32)

    def _q_of(dv):
        return (jnp.where(dv >= QN, one, zero)
                + jnp.where(dv >= 2 * QN, one, zero)
                + jnp.where(dv >= 3 * QN, one, zero))

    pltpu.sync_copy(src_hbm.at[pl.ds(w * EPT, EPT)], sbufL)
    pltpu.sync_copy(et_hbm.at[pl.ds(w * EPT, EPT)], ebufL)
    pltpu.sync_copy(dst_hbm.at[pl.ds(w * EPT, EPT)], dbufL)

    def _hist(i, cnts):
        for g in range(CH // 16):
            qv = _q_of(dbufL[pl.ds(i * CH + g * 16, 16)])
            cnts = tuple(cnts[q] + plsc.all_reduce_population_count(qv == q)
                         for q in range(4))
        return cnts

    cntv = lax.fori_loop(0, EPT // CH, _hist, (zero,) * 4)
    cnts = [_vsum(jnp.where(lanes == 0, cntv[q], 0)) for q in range(4)]
    padded = [((cnts[q] + CH - 1) // CH) * CH for q in range(4)]
    starts = [jnp.int32(0)]
    for q in range(3):
        starts.append(starts[q] + padded[q])
    trips = [padded[q] // CH for q in range(4)]
    wbase = w * CAP
    sets = ((givb0, cblb0, posb0, semg0, semc0),
            (givb1, cblb1, posb1, semg1, semc1))

    def _chunk(i, cur, bufs):
        givb, cblb, posb, semg, semc = bufs
        for g in range(CH // 16):
            base = i * CH + g * 16
            sv = sbufL[pl.ds(base, 16)]
            ev = ebufL[pl.ds(base, 16)]
            dv = dbufL[pl.ds(base, 16)]
            qv = _q_of(dv)
            giv = ev * N + sv
            cblv = dv * R + ev - qv * QROWS
            pos = zero
            ncur = []
            for q in range(4):
                m = qv == q
                mi = jnp.where(m, one, zero)
                rk = plsc.cumsum(mi)
                pos = jnp.where(m, (wbase + starts[q]) + cur[q] + rk - 1, pos)
                ncur.append(cur[q] + plsc.all_reduce_population_count(m))
            givb[pl.ds(g * 16, 16)] = giv
            cblb[pl.ds(g * 16, 16)] = cblv
            posb[pl.ds(g * 16, 16)] = pos
            cur = tuple(ncur)
        cg = pltpu.async_copy(givb, gi_hbm.at[posb], semg)
        cc = pltpu.async_copy(cblb, cbl_hbm.at[posb], semc)
        return cur, (cg, cc)

    def _pair(j, cur):
        cur, h0 = _chunk(2 * j, cur, sets[0])
        cur, h1 = _chunk(2 * j + 1, cur, sets[1])
        for h in (*h0, *h1):
            h.wait()
        return cur

    npair = (EPT // CH) // 2
    cur = lax.fori_loop(0, npair, _pair, (zero,) * 4)
    for i in range(2 * npair, EPT // CH):
        cur, hs = _chunk(i, cur, sets[0])
        for h in hs:
            h.wait()
    cur = [_vsum(jnp.where(lanes == 0, cur[q], 0)) for q in range(4)]

    for q in range(4):
        padcnt = padded[q] - cur[q]
        for j in range(CH // 16):
            m = (lanes + j * 16) < padcnt
            pos = jnp.where(m, wbase + starts[q] + cur[q] + j * 16 + lanes,
                            wbase + TRASH_OFF + lanes)
            givp[pl.ds(0, 16)] = lanes + q * 16
            cblp[pl.ds(0, 16)] = QROWS + w * 16 + lanes
            posp[pl.ds(0, 16)] = pos
            pltpu.sync_copy(givp, gi_hbm.at[posp])
            pltpu.sync_copy(cblp, cbl_hbm.at[posp])

    mv = zero
    for q in range(4):
        mv = mv + jnp.where(lanes == q, one, zero) * starts[q]
        mv = mv + jnp.where(lanes == 4 + q, one, zero) * trips[q]
    mbuf[pl.ds(0, 16)] = mv
    pltpu.sync_copy(mbuf, meta_hbm.at[pl.ds(w * 16, 16)])


@_sc_kernel(
    out_type=jax.ShapeDtypeStruct((NRP,), jnp.float32),
    scratch_types=[
        pltpu.VMEM_SHARED((NRP,), jnp.float32),
        pltpu.VMEM((2 * CH,), jnp.int32),
        pltpu.VMEM((2 * CH,), jnp.int32),
        pltpu.VMEM((CH,), jnp.int32),
        pltpu.VMEM((CH,), jnp.int32),
        pltpu.VMEM((CH,), jnp.float32),
        pltpu.VMEM((128,), jnp.float32),
        pltpu.VMEM((128,), jnp.float32),
        pltpu.VMEM((128,), jnp.float32),
        pltpu.SemaphoreType.DMA,
        pltpu.SemaphoreType.DMA,
    ],
)
def _s1(dst_hbm, et_hbm, norm_hbm, cnt, dbuf, ebuf, cbuf, cbuf2, ones,
        zeros, cstage, nstage, sem0, sem1):
    s = lax.axis_index("s")
    for g in range(CH // 16):
        ones[pl.ds(g * 16, 16)] = jnp.ones((16,), jnp.float32)
    for g in range(8):
        zeros[pl.ds(g * 16, 16)] = jnp.zeros((16,), jnp.float32)
    stripe = NRP // 16
    for j in range(stripe // 128):
        pltpu.sync_copy(zeros, cnt.at[pl.ds(s * stripe + j * 128, 128)])
    plsc.subcore_barrier()
    per_tile = E // 16

    def _cnt_body(i, carry):
        b = s * per_tile + i * (2 * CH)
        pltpu.sync_copy(dst_hbm.at[pl.ds(b, 2 * CH)], dbuf)
        pltpu.sync_copy(et_hbm.at[pl.ds(b, 2 * CH)], ebuf)
        for g in range(2 * CH // 16):
            dv = dbuf[pl.ds(g * 16, 16)]
            ev = ebuf[pl.ds(g * 16, 16)]
            tgt = cbuf if g < CH // 16 else cbuf2
            tgt[pl.ds((g % (CH // 16)) * 16, 16)] = dv * R + ev
        c0 = pltpu.async_copy(ones, cnt.at[cbuf], sem0, add=True)
        c1 = pltpu.async_copy(ones, cnt.at[cbuf2], sem1, add=True)
        c0.wait()
        c1.wait()
        return carry

    lax.fori_loop(0, per_tile // (2 * CH), _cnt_body, 0)
    plsc.subcore_barrier()
    for j in range(stripe // 128):
        k0 = s * stripe + j * 128
        pltpu.sync_copy(cnt.at[pl.ds(k0, 128)], cstage)
        for g in range(8):
            cv = cstage[pl.ds(g * 16, 16)]
            nstage[pl.ds(g * 16, 16)] = 1.0 / jnp.maximum(cv, 1.0)
        pltpu.sync_copy(nstage, norm_hbm.at[pl.ds(k0, 128)])


@_sc_kernel(
    out_type=jax.ShapeDtypeStruct((2, NRP, D), jnp.float32),
    scratch_types=[
        pltpu.VMEM_SHARED((ACCROWS, D), jnp.float32),
        pltpu.VMEM((CH,), jnp.int32),
        pltpu.VMEM((CH,), jnp.int32),
        pltpu.VMEM((CH, D), jnp.float32),
        pltpu.VMEM((16, D), jnp.float32),
        pltpu.VMEM((16,), jnp.int32),
        pltpu.SemaphoreType.DMA,
    ],
)
def _s2(gi_hbm, cbl_hbm, meta_hbm, h_hbm, p_hbm,
        acc, gbuf, cbuf, rowbuf, zrow, mbuf, sem):
    c = lax.axis_index("c")
    s = lax.axis_index("s")
    w = s * 2 + c
    for j in range(16):
        for g in range(D // 16):
            zrow[j, pl.ds(g * 16, 16)] = jnp.zeros((16,), jnp.float32)
    pltpu.sync_copy(meta_hbm.at[pl.ds(w * 16, 16)], mbuf)
    mv = mbuf[pl.ds(0, 16)]
    lanes = lax.iota(jnp.int32, 16)
    starts = [pl.multiple_of(_vsum(jnp.where(lanes == q, mv, 0)), CH)
              for q in range(4)]
    trips = [_vsum(jnp.where(lanes == 4 + q, mv, 0)) for q in range(4)]
    zstripe = ACCROWS // 16
    fstripe = QROWS // 16
    for q in range(4):
        for j in range(zstripe // 16):
            pltpu.sync_copy(zrow, acc.at[pl.ds(s * zstripe + j * 16, 16), :])
        plsc.subcore_barrier()

        def _edge(i, carry, q=q):
            b = w * CAP + starts[q] + i * CH
            pltpu.sync_copy(gi_hbm.at[pl.ds(b, CH)], gbuf)
            pltpu.sync_copy(cbl_hbm.at[pl.ds(b, CH)], cbuf)
            pltpu.async_copy(h_hbm.at[gbuf], rowbuf, sem).wait()
            pltpu.sync_copy(rowbuf, acc.at[cbuf], add=True)
            return carry

        lax.fori_loop(0, trips[q], _edge, 0)
        plsc.subcore_barrier()
        pltpu.sync_copy(
            acc.at[pl.ds(s * fstripe, fstripe), :],
            p_hbm.at[c, pl.ds(q * QROWS + s * fstripe, fstripe), :])
        plsc.subcore_barrier()


def _full(shape):
    return pl.BlockSpec(shape, lambda i: tuple(0 for _ in shape))


def _tc1_body(tw_ref, wt_ref, bt_ref, win_ref, bin_ref, wrel_ref, wroot_ref,
              h_ref, root_ref):
    t = _leaky(jnp.dot(tw_ref[...], wt_ref[...],
                       preferred_element_type=jnp.float32) + bt_ref[...])
    x = _leaky(jnp.dot(t, win_ref[...],
                       preferred_element_type=jnp.float32) + bin_ref[...])
    root_ref[...] = jnp.dot(x, wroot_ref[...],
                            preferred_element_type=jnp.float32)
    for r in range(R):
        h_ref[r] = jnp.dot(x, wrel_ref[r], preferred_element_type=jnp.float32)


def _tc1(tweet, W_t, b_t, W_in, b_in, W_rel, W_root):
    return pl.pallas_call(
        _tc1_body,
        grid=(GRID,),
        in_specs=[
            pl.BlockSpec((BN, TS), lambda i: (i, 0)),
            _full((TS, D)), _full((1, D)), _full((D, D)), _full((1, D)),
            _full((R, D, D)), _full((D, D)),
        ],
        out_specs=[
            pl.BlockSpec((R, BN, D), lambda i: (0, i, 0)),
            pl.BlockSpec((BN, D), lambda i: (i, 0)),
        ],
        out_shape=[
            jax.ShapeDtypeStruct((R, N, D), jnp.float32),
            jax.ShapeDtypeStruct((N, D), jnp.float32),
        ],
    )(tweet, W_t, b_t.reshape(1, D), W_in, b_in.reshape(1, D), W_rel, W_root)


def _combine(p_ref, nrm_ref, root_ref, brg_ref):
    pm = p_ref[0] + p_ref[1]
    pm = pm.reshape(BN, R, D)
    nrm = nrm_ref[...]
    agg = jnp.sum(pm * nrm[:, :, None], axis=1)
    return agg + root_ref[...] + brg_ref[...]


def _tc2_body(p_ref, nrm_ref, root_ref, brg_ref, wrel_ref, wroot_ref,
              h_ref, rootb_ref):
    x = _combine(p_ref, nrm_ref, root_ref, brg_ref)
    rootb_ref[...] = jnp.dot(x, wroot_ref[...],
                             preferred_element_type=jnp.float32)
    for r in range(R):
        h_ref[r] = jnp.dot(x, wrel_ref[r], preferred_element_type=jnp.float32)


def _tc2(P, norm5, rootA, b_rgcn, W_rel, W_root):
    return pl.pallas_call(
        _tc2_body,
        grid=(GRID,),
        in_specs=[
            pl.BlockSpec((2, BN * R, D), lambda i: (0, i, 0)),
            pl.BlockSpec((BN, R), lambda i: (i, 0)),
            pl.BlockSpec((BN, D), lambda i: (i, 0)),
            _full((1, D)), _full((R, D, D)), _full((D, D)),
        ],
        out_specs=[
            pl.BlockSpec((R, BN, D), lambda i: (0, i, 0)),
            pl.BlockSpec((BN, D), lambda i: (i, 0)),
        ],
        out_shape=[
            jax.ShapeDtypeStruct((R, N, D), jnp.float32),
            jax.ShapeDtypeStruct((N, D), jnp.float32),
        ],
    )(P, norm5, rootA, b_rgcn.reshape(1, D), W_rel, W_root)


def _tc3_body(p_ref, nrm_ref, root_ref, brg_ref, wo1_ref, bo1_ref,
              wo2_ref, bo2_ref, out_ref):
    x = _combine(p_ref, nrm_ref, root_ref, brg_ref)
    x = _leaky(jnp.dot(x, wo1_ref[...],
                       preferred_element_type=jnp.float32) + bo1_ref[...])
    out_ref[...] = jnp.dot(x, wo2_ref[...],
                           preferred_element_type=jnp.float32) + bo2_ref[...]


def _tc3(P, norm5, rootB, b_rgcn, W_o1, b_o1, W_o2p, b_o2p):
    return pl.pallas_call(
        _tc3_body,
        grid=(GRID,),
        in_specs=[
            pl.BlockSpec((2, BN * R, D), lambda i: (0, i, 0)),
            pl.BlockSpec((BN, R), lambda i: (i, 0)),
            pl.BlockSpec((BN, D), lambda i: (i, 0)),
            _full((1, D)), _full((D, D)), _full((1, D)),
            _full((D, D)), _full((1, D)),
        ],
        out_specs=pl.BlockSpec((BN, D), lambda i: (i, 0)),
        out_shape=jax.ShapeDtypeStruct((N, D), jnp.float32),
    )(P, norm5, rootB, b_rgcn.reshape(1, D), W_o1, b_o1.reshape(1, D),
      W_o2p, b_o2p.reshape(1, D))


def kernel(des, tweet, num_prop, cat_prop, edge_index, edge_type,
           W_t, b_t, W_in, b_in, W_rel, W_root, b_rgcn, W_o1, b_o1,
           W_o2, b_o2):
    src = edge_index[0]
    dst = edge_index[1]
    et = edge_type.astype(jnp.int32)

    gi, cbl, meta = _part(src, et, dst)
    norm_flat = _s1(dst, et)
    norm5 = norm_flat.reshape(NP, R)

    hA, rootA = _tc1(tweet, W_t, b_t, W_in, b_in, W_rel, W_root)
    PA = _s2(gi, cbl, meta, hA.reshape(R * N, D))
    hB, rootB = _tc2(PA, norm5, rootA, b_rgcn, W_rel, W_root)
    PB = _s2(gi, cbl, meta, hB.reshape(R * N, D))

    W_o2p = jnp.zeros((D, D), jnp.float32).at[:, :2].set(W_o2)
    b_o2p = jnp.zeros((D,), jnp.float32).at[:2].set(b_o2)
    out_full = _tc3(PB, norm5, rootB, b_rgcn, W_o1, b_o1, W_o2p, b_o2p)
    return out_full[:, :2]


# speedup vs baseline: 1.0430x; 1.0430x over previous
import functools

import jax
import jax.numpy as jnp
from jax import lax
from jax.experimental import pallas as pl
from jax.experimental.pallas import tpu as pltpu, tpu_sc as plsc

N = 10000
E = 320000
R = 5
D = 128
TS = 768

NRP = 51200
NP = NRP // R
QROWS = NRP // 4
QN = NP // 4
ACCROWS = 13312
CH = 80
CAPB = 10080
CAP = 10336
TRASH_OFF = 10320
TOT = 32 * CAP
BN = 400
GRID = N // BN
EPT = E // 32


def _leaky(x):
    return jnp.where(x >= 0, x, 0.01 * x)


def _vsum(v):
    return lax.reduce_sum_p.bind(v, axes=(0,))


def _isum(m):
    v = jnp.where(m, jnp.full((16,), 1, jnp.int32),
                  jnp.full((16,), 0, jnp.int32))
    return lax.reduce_sum_p.bind(v, axes=(0,))


def _sc_kernel(**kw):
    def deco(fn):
        @functools.wraps(fn)
        def call(*args):
            mesh = plsc.VectorSubcoreMesh(core_axis_name="c",
                                          subcore_axis_name="s")
            params = pltpu.CompilerParams(needs_layout_passes=False)
            return pl.kernel(fn, mesh=mesh, compiler_params=params,
                             **kw)(*args)
        return call
    return deco


@_sc_kernel(
    out_type=(
        jax.ShapeDtypeStruct((TOT,), jnp.int32),
        jax.ShapeDtypeStruct((TOT,), jnp.int32),
        jax.ShapeDtypeStruct((512,), jnp.int32),
    ),
    scratch_types=[
        pltpu.VMEM((EPT,), jnp.int32),
        pltpu.VMEM((EPT,), jnp.int32),
        pltpu.VMEM((EPT,), jnp.int32),
        pltpu.VMEM((CH,), jnp.int32),
        pltpu.VMEM((CH,), jnp.int32),
        pltpu.VMEM((CH,), jnp.int32),
        pltpu.VMEM((CH,), jnp.int32),
        pltpu.VMEM((CH,), jnp.int32),
        pltpu.VMEM((CH,), jnp.int32),
        pltpu.VMEM((16,), jnp.int32),
        pltpu.VMEM((16,), jnp.int32),
        pltpu.VMEM((16,), jnp.int32),
        pltpu.VMEM((16,), jnp.int32),
        pltpu.SemaphoreType.DMA,
        pltpu.SemaphoreType.DMA,
        pltpu.SemaphoreType.DMA,
        pltpu.SemaphoreType.DMA,
    ],
)
def _part(src_hbm, et_hbm, dst_hbm, gi_hbm, cbl_hbm, meta_hbm,
          sbufL, ebufL, dbufL, givb0, cblb0, posb0, givb1, cblb1, posb1,
          givp, cblp, posp, mbuf, semg0, semc0, semg1, semc1):
    c = lax.axis_index("c")
    s = lax.axis_index("s")
    w = s * 2 + c
    lanes = lax.iota(jnp.int32, 16)
    one = jnp.full((16,), 1, jnp.int32)
    zero = jnp.full((16,), 0, jnp.int32)

    def _q_of(dv):
        return (jnp.where(dv >= QN, one, zero)
                + jnp.where(dv >= 2 * QN, one, zero)
                + jnp.where(dv >= 3 * QN, one, zero))

    pltpu.sync_copy(src_hbm.at[pl.ds(w * EPT, EPT)], sbufL)
    pltpu.sync_copy(et_hbm.at[pl.ds(w * EPT, EPT)], ebufL)
    pltpu.sync_copy(dst_hbm.at[pl.ds(w * EPT, EPT)], dbufL)

    def _hist(i, cnts):
        for g in range(CH // 16):
            qv = _q_of(dbufL[pl.ds(i * CH + g * 16, 16)])
            cnts = tuple(cnts[q] + _isum(qv == q) for q in range(4))
        return cnts

    cnts = lax.fori_loop(0, EPT // CH, _hist, (jnp.int32(0),) * 4)
    padded = [((cnts[q] + CH - 1) // CH) * CH for q in range(4)]
    starts = [jnp.int32(0)]
    for q in range(3):
        starts.append(starts[q] + padded[q])
    trips = [padded[q] // CH for q in range(4)]
    wbase = w * CAP
    sets = ((givb0, cblb0, posb0, semg0, semc0),
            (givb1, cblb1, posb1, semg1, semc1))

    def _chunk(i, cur, bufs):
        givb, cblb, posb, semg, semc = bufs
        for g in range(CH // 16):
            base = i * CH + g * 16
            sv = sbufL[pl.ds(base, 16)]
            ev = ebufL[pl.ds(base, 16)]
            dv = dbufL[pl.ds(base, 16)]
            qv = _q_of(dv)
            giv = ev * N + sv
            cblv = dv * R + ev - qv * QROWS
            pos = zero
            ncur = []
            for q in range(4):
                m = qv == q
                mi = jnp.where(m, one, zero)
                rk = plsc.cumsum(mi)
                pos = jnp.where(m, wbase + starts[q] + cur[q] + rk - 1, pos)
                ncur.append(cur[q] + _vsum(mi))
            givb[pl.ds(g * 16, 16)] = giv
            cblb[pl.ds(g * 16, 16)] = cblv
            posb[pl.ds(g * 16, 16)] = pos
            cur = tuple(ncur)
        cg = pltpu.async_copy(givb, gi_hbm.at[posb], semg)
        cc = pltpu.async_copy(cblb, cbl_hbm.at[posb], semc)
        return cur, (cg, cc)

    def _pair(j, cur):
        cur, h0 = _chunk(2 * j, cur, sets[0])
        cur, h1 = _chunk(2 * j + 1, cur, sets[1])
        for h in (*h0, *h1):
            h.wait()
        return cur

    npair = (EPT // CH) // 2
    cur = lax.fori_loop(0, npair, _pair, (jnp.int32(0),) * 4)
    for i in range(2 * npair, EPT // CH):
        cur, hs = _chunk(i, cur, sets[0])
        for h in hs:
            h.wait()

    for q in range(4):
        padcnt = padded[q] - cur[q]
        for j in range(CH // 16):
            m = (lanes + j * 16) < padcnt
            pos = jnp.where(m, wbase + starts[q] + cur[q] + j * 16 + lanes,
                            wbase + TRASH_OFF + lanes)
            givp[pl.ds(0, 16)] = lanes + q * 16
            cblp[pl.ds(0, 16)] = QROWS + w * 16 + lanes
            posp[pl.ds(0, 16)] = pos
            pltpu.sync_copy(givp, gi_hbm.at[posp])
            pltpu.sync_copy(cblp, cbl_hbm.at[posp])

    mv = zero
    for q in range(4):
        mv = mv + jnp.where(lanes == q, one, zero) * starts[q]
        mv = mv + jnp.where(lanes == 4 + q, one, zero) * trips[q]
    mbuf[pl.ds(0, 16)] = mv
    pltpu.sync_copy(mbuf, meta_hbm.at[pl.ds(w * 16, 16)])


@_sc_kernel(
    out_type=jax.ShapeDtypeStruct((NRP,), jnp.float32),
    scratch_types=[
        pltpu.VMEM_SHARED((NRP,), jnp.float32),
        pltpu.VMEM((2 * CH,), jnp.int32),
        pltpu.VMEM((2 * CH,), jnp.int32),
        pltpu.VMEM((CH,), jnp.int32),
        pltpu.VMEM((CH,), jnp.int32),
        pltpu.VMEM((CH,), jnp.float32),
        pltpu.VMEM((128,), jnp.float32),
        pltpu.VMEM((128,), jnp.float32),
        pltpu.VMEM((128,), jnp.float32),
        pltpu.SemaphoreType.DMA,
        pltpu.SemaphoreType.DMA,
    ],
)
def _s1(dst_hbm, et_hbm, norm_hbm, cnt, dbuf, ebuf, cbuf, cbuf2, ones,
        zeros, cstage, nstage, sem0, sem1):
    s = lax.axis_index("s")
    for g in range(CH // 16):
        ones[pl.ds(g * 16, 16)] = jnp.ones((16,), jnp.float32)
    for g in range(8):
        zeros[pl.ds(g * 16, 16)] = jnp.zeros((16,), jnp.float32)
    stripe = NRP // 16
    for j in range(stripe // 128):
        pltpu.sync_copy(zeros, cnt.at[pl.ds(s * stripe + j * 128, 128)])
    plsc.subcore_barrier()
    per_tile = E // 16

    def _cnt_body(i, carry):
        b = s * per_tile + i * (2 * CH)
        pltpu.sync_copy(dst_hbm.at[pl.ds(b, 2 * CH)], dbuf)
        pltpu.sync_copy(et_hbm.at[pl.ds(b, 2 * CH)], ebuf)
        for g in range(2 * CH // 16):
            dv = dbuf[pl.ds(g * 16, 16)]
            ev = ebuf[pl.ds(g * 16, 16)]
            tgt = cbuf if g < CH // 16 else cbuf2
            tgt[pl.ds((g % (CH // 16)) * 16, 16)] = dv * R + ev
        c0 = pltpu.async_copy(ones, cnt.at[cbuf], sem0, add=True)
        c1 = pltpu.async_copy(ones, cnt.at[cbuf2], sem1, add=True)
        c0.wait()
        c1.wait()
        return carry

    lax.fori_loop(0, per_tile // (2 * CH), _cnt_body, 0)
    plsc.subcore_barrier()
    for j in range(stripe // 128):
        k0 = s * stripe + j * 128
        pltpu.sync_copy(cnt.at[pl.ds(k0, 128)], cstage)
        for g in range(8):
            cv = cstage[pl.ds(g * 16, 16)]
            nstage[pl.ds(g * 16, 16)] = 1.0 / jnp.maximum(cv, 1.0)
        pltpu.sync_copy(nstage, norm_hbm.at[pl.ds(k0, 128)])


@_sc_kernel(
    out_type=jax.ShapeDtypeStruct((2, NRP, D), jnp.float32),
    scratch_types=[
        pltpu.VMEM_SHARED((ACCROWS, D), jnp.float32),
        pltpu.VMEM((CH,), jnp.int32),
        pltpu.VMEM((CH,), jnp.int32),
        pltpu.VMEM((CH, D), jnp.float32),
        pltpu.VMEM((16, D), jnp.float32),
        pltpu.VMEM((16,), jnp.int32),
        pltpu.SemaphoreType.DMA,
    ],
)
def _s2(gi_hbm, cbl_hbm, meta_hbm, h_hbm, p_hbm,
        acc, gbuf, cbuf, rowbuf, zrow, mbuf, sem):
    c = lax.axis_index("c")
    s = lax.axis_index("s")
    w = s * 2 + c
    for j in range(16):
        for g in range(D // 16):
            zrow[j, pl.ds(g * 16, 16)] = jnp.zeros((16,), jnp.float32)
    pltpu.sync_copy(meta_hbm.at[pl.ds(w * 16, 16)], mbuf)
    mv = mbuf[pl.ds(0, 16)]
    lanes = lax.iota(jnp.int32, 16)
    starts = [pl.multiple_of(_vsum(jnp.where(lanes == q, mv, 0)), CH)
              for q in range(4)]
    trips = [_vsum(jnp.where(lanes == 4 + q, mv, 0)) for q in range(4)]
    zstripe = ACCROWS // 16
    fstripe = QROWS // 16
    for q in range(4):
        for j in range(zstripe // 16):
            pltpu.sync_copy(zrow, acc.at[pl.ds(s * zstripe + j * 16, 16), :])
        plsc.subcore_barrier()

        def _edge(i, carry, q=q):
            b = w * CAP + starts[q] + i * CH
            pltpu.sync_copy(gi_hbm.at[pl.ds(b, CH)], gbuf)
            pltpu.sync_copy(cbl_hbm.at[pl.ds(b, CH)], cbuf)
            pltpu.async_copy(h_hbm.at[gbuf], rowbuf, sem).wait()
            pltpu.sync_copy(rowbuf, acc.at[cbuf], add=True)
            return carry

        lax.fori_loop(0, trips[q], _edge, 0)
        plsc.subcore_barrier()
        pltpu.sync_copy(
            acc.at[pl.ds(s * fstripe, fstripe), :],
            p_hbm.at[c, pl.ds(q * QROWS + s * fstripe, fstripe), :])
        plsc.subcore_barrier()


def _full(shape):
    return pl.BlockSpec(shape, lambda i: tuple(0 for _ in shape))


def _tc1_body(tw_ref, wt_ref, bt_ref, win_ref, bin_ref, wrel_ref, wroot_ref,
              h_ref, root_ref):
    t = _leaky(jnp.dot(tw_ref[...], wt_ref[...],
                       preferred_element_type=jnp.float32) + bt_ref[...])
    x = _leaky(jnp.dot(t, win_ref[...],
                       preferred_element_type=jnp.float32) + bin_ref[...])
    root_ref[...] = jnp.dot(x, wroot_ref[...],
                            preferred_element_type=jnp.float32)
    for r in range(R):
        h_ref[r] = jnp.dot(x, wrel_ref[r], preferred_element_type=jnp.float32)


def _tc1(tweet, W_t, b_t, W_in, b_in, W_rel, W_root):
    return pl.pallas_call(
        _tc1_body,
        grid=(GRID,),
        in_specs=[
            pl.BlockSpec((BN, TS), lambda i: (i, 0)),
            _full((TS, D)), _full((1, D)), _full((D, D)), _full((1, D)),
            _full((R, D, D)), _full((D, D)),
        ],
        out_specs=[
            pl.BlockSpec((R, BN, D), lambda i: (0, i, 0)),
            pl.BlockSpec((BN, D), lambda i: (i, 0)),
        ],
        out_shape=[
            jax.ShapeDtypeStruct((R, N, D), jnp.float32),
            jax.ShapeDtypeStruct((N, D), jnp.float32),
        ],
    )(tweet, W_t, b_t.reshape(1, D), W_in, b_in.reshape(1, D), W_rel, W_root)


def _combine(p_ref, nrm_ref, root_ref, brg_ref):
    pm = p_ref[0] + p_ref[1]
    pm = pm.reshape(BN, R, D)
    nrm = nrm_ref[...]
    agg = jnp.sum(pm * nrm[:, :, None], axis=1)
    return agg + root_ref[...] + brg_ref[...]


def _tc2_body(p_ref, nrm_ref, root_ref, brg_ref, wrel_ref, wroot_ref,
              h_ref, rootb_ref):
    x = _combine(p_ref, nrm_ref, root_ref, brg_ref)
    rootb_ref[...] = jnp.dot(x, wroot_ref[...],
                             preferred_element_type=jnp.float32)
    for r in range(R):
        h_ref[r] = jnp.dot(x, wrel_ref[r], preferred_element_type=jnp.float32)


def _tc2(P, norm5, rootA, b_rgcn, W_rel, W_root):
    return pl.pallas_call(
        _tc2_body,
        grid=(GRID,),
        in_specs=[
            pl.BlockSpec((2, BN * R, D), lambda i: (0, i, 0)),
            pl.BlockSpec((BN, R), lambda i: (i, 0)),
            pl.BlockSpec((BN, D), lambda i: (i, 0)),
            _full((1, D)), _full((R, D, D)), _full((D, D)),
        ],
        out_specs=[
            pl.BlockSpec((R, BN, D), lambda i: (0, i, 0)),
            pl.BlockSpec((BN, D), lambda i: (i, 0)),
        ],
        out_shape=[
            jax.ShapeDtypeStruct((R, N, D), jnp.float32),
            jax.ShapeDtypeStruct((N, D), jnp.float32),
        ],
    )(P, norm5, rootA, b_rgcn.reshape(1, D), W_rel, W_root)


def _tc3_body(p_ref, nrm_ref, root_ref, brg_ref, wo1_ref, bo1_ref,
              wo2_ref, bo2_ref, out_ref):
    x = _combine(p_ref, nrm_ref, root_ref, brg_ref)
    x = _leaky(jnp.dot(x, wo1_ref[...],
                       preferred_element_type=jnp.float32) + bo1_ref[...])
    out_ref[...] = jnp.dot(x, wo2_ref[...],
                           preferred_element_type=jnp.float32) + bo2_ref[...]


def _tc3(P, norm5, rootB, b_rgcn, W_o1, b_o1, W_o2p, b_o2p):
    return pl.pallas_call(
        _tc3_body,
        grid=(GRID,),
        in_specs=[
            pl.BlockSpec((2, BN * R, D), lambda i: (0, i, 0)),
            pl.BlockSpec((BN, R), lambda i: (i, 0)),
            pl.BlockSpec((BN, D), lambda i: (i, 0)),
            _full((1, D)), _full((D, D)), _full((1, D)),
            _full((D, D)), _full((1, D)),
        ],
        out_specs=pl.BlockSpec((BN, D), lambda i: (i, 0)),
        out_shape=jax.ShapeDtypeStruct((N, D), jnp.float32),
    )(P, norm5, rootB, b_rgcn.reshape(1, D), W_o1, b_o1.reshape(1, D),
      W_o2p, b_o2p.reshape(1, D))


def kernel(des, tweet, num_prop, cat_prop, edge_index, edge_type,
           W_t, b_t, W_in, b_in, W_rel, W_root, b_rgcn, W_o1, b_o1,
           W_o2, b_o2):
    src = edge_index[0]
    dst = edge_index[1]
    et = edge_type.astype(jnp.int32)

    gi, cbl, meta = _part(src, et, dst)
    norm_flat = _s1(dst, et)
    norm5 = norm_flat.reshape(NP, R)

    hA, rootA = _tc1(tweet, W_t, b_t, W_in, b_in, W_rel, W_root)
    PA = _s2(gi, cbl, meta, hA.reshape(R * N, D))
    hB, rootB = _tc2(PA, norm5, rootA, b_rgcn, W_rel, W_root)
    PB = _s2(gi, cbl, meta, hB.reshape(R * N, D))

    W_o2p = jnp.zeros((D, D), jnp.float32).at[:, :2].set(W_o2)
    b_o2p = jnp.zeros((D,), jnp.float32).at[:2].set(b_o2)
    out_full = _tc3(PB, norm5, rootB, b_rgcn, W_o1, b_o1, W_o2p, b_o2p)
    return out_full[:, :2]
